# baseline (speedup 1.0000x reference)
import jax
import jax.numpy as jnp
from jax.experimental import pallas as pl
from jax.experimental.pallas import tpu as pltpu

B, S, IN = 8, 96, 128
H, M, N = 2, 32, 512
STATE, OUT, NS = 256, 126, 3
EPS = 1e-12
CIN = IN + H * M + STATE
PHEAD = NS + 1 + 3 + 1 + M + M + M + 1 + 1
TOT = STATE + OUT + H * PHEAD
f32 = jnp.float32
bf16 = jnp.bfloat16


def _roll_m1(x):
    return jnp.concatenate([x[:, 1:], x[:, :1]], axis=-1)


def _roll_p1(x):
    return jnp.concatenate([x[:, -1:], x[:, :-1]], axis=-1)


def _dwm_kernel(x_ref, w_ref, b_ref, out_ref, mem_ref):
    a0 = (jax.lax.broadcasted_iota(jnp.int32, (B, N), 1) == 0).astype(f32)
    mem_ref[...] = jnp.full((B, M, N), 0.01, f32)

    def step(t, carry):
        state, wt0, wt1, wd0, wd1, read0, read1 = carry
        x_t = x_ref[pl.ds(t, 1)].reshape(B, IN)
        comb = jnp.concatenate([x_t, read0, read1, state], axis=-1)
        res = jnp.dot(comb.astype(bf16), w_ref[...],
                      preferred_element_type=f32) + b_ref[...]
        state_n = jax.nn.sigmoid(res[:, :STATE])
        out_ref[pl.ds(t, 1)] = res[:, STATE:STATE + OUT].reshape(1, B, OUT)

        P = STATE + OUT
        pr = []
        for h in range(H):
            r = res[:, P + h * PHEAD:P + (h + 1) * PHEAD]
            pr.append(dict(
                s=jax.nn.softmax(jax.nn.softplus(r[:, 0:3]), axis=-1),
                jd=jax.nn.sigmoid(r[:, 3:4]),
                j=jax.nn.softmax(r[:, 4:7], axis=-1),
                gamma=1.0 + jax.nn.softplus(r[:, 7:8]),
                erase=jax.nn.sigmoid(r[:, 8:8 + M]),
                add=r[:, 8 + M:8 + 2 * M],
                k=jnp.tanh(r[:, 8 + 2 * M:8 + 3 * M]),
                beta=jax.nn.softplus(r[:, 104:105]),
                g=jax.nn.sigmoid(r[:, 105:106]),
            ))

        mem = mem_ref[...]
        f0 = 1.0 - pr[0]["erase"][:, :, None] * wt0[:, None, :]
        f1 = 1.0 - pr[1]["erase"][:, :, None] * wt1[:, None, :]
        mem = mem * (f0 * f1) \
            + pr[0]["add"][:, :, None] * wt0[:, None, :] \
            + pr[1]["add"][:, :, None] * wt1[:, None, :]
        mem_ref[...] = mem

        denom = jnp.sqrt(jnp.sum(mem * mem, axis=1)) + EPS
        wts_new = []
        wds_new = []
        reads_new = []
        for h, wt, wd in ((0, wt0, wd0), (1, wt1, wd1)):
            p = pr[h]
            kk = p["k"]
            kn = kk / (jnp.sqrt(jnp.sum(kk * kk, axis=-1, keepdims=True)) + EPS)
            wtk = jnp.sum(kn[:, :, None] * mem, axis=1) / denom
            e = jnp.exp(p["beta"] * wtk)
            sm = e / jnp.sum(e, axis=-1, keepdims=True)
            wtc = p["g"] * sm + (1.0 - p["g"]) * wt
            conv = p["s"][:, 0:1] * _roll_m1(wtc) \
                + p["s"][:, 1:2] * wtc \
                + p["s"][:, 2:3] * _roll_p1(wtc)
            wt_sh = jnp.exp2(p["gamma"] * jnp.log2(conv + EPS))
            swt = jnp.sum(wt_sh, axis=-1, keepdims=True)
            wtn = wt_sh / swt
            wd_n = (1.0 - p["jd"]) * wd + p["jd"] * wtn
            wt_f = p["j"][:, 0:1] * wtn + p["j"][:, 1:2] * a0 + p["j"][:, 2:3] * wd
            rd_sh = jnp.sum(wt_sh[:, None, :] * mem, axis=-1)
            rd_wd = jnp.sum(wd[:, None, :] * mem, axis=-1)
            read_n = p["j"][:, 0:1] * rd_sh / swt \
                + p["j"][:, 1:2] * mem[:, :, 0] \
                + p["j"][:, 2:3] * rd_wd
            wts_new.append(wt_f)
            wds_new.append(wd_n)
            reads_new.append(read_n)

        return (state_n, wts_new[0], wts_new[1], wds_new[0], wds_new[1],
                reads_new[0], reads_new[1])

    r0 = jnp.full((B, M), 0.01, f32)
    init = (jnp.ones((B, STATE), f32), a0, a0, a0, a0, r0, r0)

    UNROLL = 2

    def body(i, carry):
        t = i * UNROLL
        for u in range(UNROLL):
            carry = step(t + u, carry)
        return carry

    jax.lax.fori_loop(0, S // UNROLL, body, init)


def kernel(x, W_s, b_s, W_o, b_o, W_u, b_u):
    w_cat = jnp.concatenate([W_s, W_o, W_u], axis=1).astype(bf16)
    b_cat = jnp.concatenate([b_s, b_o, b_u]).reshape(1, TOT)
    xT = jnp.swapaxes(x, 0, 1)
    out = pl.pallas_call(
        _dwm_kernel,
        grid=(1,),
        in_specs=[
            pl.BlockSpec((S, B, IN), lambda i: (0, 0, 0)),
            pl.BlockSpec((CIN, TOT), lambda i: (0, 0)),
            pl.BlockSpec((1, TOT), lambda i: (0, 0)),
        ],
        out_specs=pl.BlockSpec((S, B, OUT), lambda i: (0, 0, 0)),
        out_shape=jax.ShapeDtypeStruct((S, B, OUT), f32),
        scratch_shapes=[pltpu.VMEM((B, M, N), f32)],
        compiler_params=pltpu.CompilerParams(
            dimension_semantics=("arbitrary",),
        ),
    )(xT, w_cat, b_cat)
    return jnp.swapaxes(out, 0, 1)


# speedup vs baseline: 1.0750x; 1.0750x over previous
import jax
import jax.numpy as jnp
from jax.experimental import pallas as pl
from jax.experimental.pallas import tpu as pltpu

B, S, IN = 8, 96, 128
H, M, N = 2, 32, 512
STATE, OUT, NS = 256, 126, 3
EPS = 1e-12
CIN = IN + H * M + STATE
PHEAD = NS + 1 + 3 + 1 + M + M + M + 1 + 1
TOT = STATE + OUT + H * PHEAD
f32 = jnp.float32
bf16 = jnp.bfloat16


def _roll_m1(x):
    return jnp.concatenate([x[:, 1:], x[:, :1]], axis=-1)


def _roll_p1(x):
    return jnp.concatenate([x[:, -1:], x[:, :-1]], axis=-1)


def _dwm_kernel(x_ref, w_ref, b_ref, out_ref, mem_ref):
    a0 = (jax.lax.broadcasted_iota(jnp.int32, (B, N), 1) == 0).astype(f32)
    mem_ref[...] = jnp.full((B, M, N), 0.01, f32)

    def step(t, carry):
        state, wt0, wt1, wd0, wd1 = carry
        mem = mem_ref[...]
        x_t = x_ref[pl.ds(t, 1)].reshape(B, IN)
        read0 = jnp.sum(wt0[:, None, :] * mem, axis=-1)
        read1 = jnp.sum(wt1[:, None, :] * mem, axis=-1)
        comb = jnp.concatenate([x_t, read0, read1, state], axis=-1)
        res = jnp.dot(comb.astype(bf16), w_ref[...],
                      preferred_element_type=f32) + b_ref[...]
        state_n = jax.nn.sigmoid(res[:, :STATE])
        out_ref[pl.ds(t, 1)] = res[:, STATE:STATE + OUT].reshape(1, B, OUT)

        P = STATE + OUT
        pr = []
        for h in range(H):
            r = res[:, P + h * PHEAD:P + (h + 1) * PHEAD]
            pr.append(dict(
                s=jax.nn.softmax(jax.nn.softplus(r[:, 0:3]), axis=-1),
                jd=jax.nn.sigmoid(r[:, 3:4]),
                j=jax.nn.softmax(r[:, 4:7], axis=-1),
                gamma=1.0 + jax.nn.softplus(r[:, 7:8]),
                erase=jax.nn.sigmoid(r[:, 8:8 + M]),
                add=r[:, 8 + M:8 + 2 * M],
                k=jnp.tanh(r[:, 8 + 2 * M:8 + 3 * M]),
                beta=jax.nn.softplus(r[:, 104:105]),
                g=jax.nn.sigmoid(r[:, 105:106]),
            ))

        f0 = 1.0 - pr[0]["erase"][:, :, None] * wt0[:, None, :]
        f1 = 1.0 - pr[1]["erase"][:, :, None] * wt1[:, None, :]
        mem = mem * (f0 * f1) \
            + pr[0]["add"][:, :, None] * wt0[:, None, :] \
            + pr[1]["add"][:, :, None] * wt1[:, None, :]
        mem_ref[...] = mem

        denom = jnp.sqrt(jnp.sum(mem * mem, axis=1)) + EPS
        wts_new = []
        wds_new = []
        for h, wt, wd in ((0, wt0, wd0), (1, wt1, wd1)):
            p = pr[h]
            kk = p["k"]
            kn = kk / (jnp.sqrt(jnp.sum(kk * kk, axis=-1, keepdims=True)) + EPS)
            wtk = jnp.sum(kn[:, :, None] * mem, axis=1) / denom
            e = jnp.exp(p["beta"] * wtk)
            sm = e / jnp.sum(e, axis=-1, keepdims=True)
            wtc = p["g"] * sm + (1.0 - p["g"]) * wt
            conv = p["s"][:, 0:1] * _roll_m1(wtc) \
                + p["s"][:, 1:2] * wtc \
                + p["s"][:, 2:3] * _roll_p1(wtc)
            wt_sh = jnp.exp2(p["gamma"] * jnp.log2(conv + EPS))
            wtn = wt_sh / jnp.sum(wt_sh, axis=-1, keepdims=True)
            wd_n = (1.0 - p["jd"]) * wd + p["jd"] * wtn
            wt_f = p["j"][:, 0:1] * wtn + p["j"][:, 1:2] * a0 + p["j"][:, 2:3] * wd
            wts_new.append(wt_f)
            wds_new.append(wd_n)

        return (state_n, wts_new[0], wts_new[1], wds_new[0], wds_new[1])

    init = (jnp.ones((B, STATE), f32), a0, a0, a0, a0)

    UNROLL = 2

    def body(i, carry):
        t = i * UNROLL
        for u in range(UNROLL):
            carry = step(t + u, carry)
        return carry

    jax.lax.fori_loop(0, S // UNROLL, body, init)


def kernel(x, W_s, b_s, W_o, b_o, W_u, b_u):
    w_cat = jnp.concatenate([W_s, W_o, W_u], axis=1).astype(bf16)
    b_cat = jnp.concatenate([b_s, b_o, b_u]).reshape(1, TOT)
    xT = jnp.swapaxes(x, 0, 1)
    out = pl.pallas_call(
        _dwm_kernel,
        grid=(1,),
        in_specs=[
            pl.BlockSpec((S, B, IN), lambda i: (0, 0, 0)),
            pl.BlockSpec((CIN, TOT), lambda i: (0, 0)),
            pl.BlockSpec((1, TOT), lambda i: (0, 0)),
        ],
        out_specs=pl.BlockSpec((S, B, OUT), lambda i: (0, 0, 0)),
        out_shape=jax.ShapeDtypeStruct((S, B, OUT), f32),
        scratch_shapes=[pltpu.VMEM((B, M, N), f32)],
        compiler_params=pltpu.CompilerParams(
            dimension_semantics=("arbitrary",),
        ),
    )(xT, w_cat, b_cat)
    return jnp.swapaxes(out, 0, 1)
